# baseline (speedup 1.0000x reference)
import functools
import math

import jax
import jax.numpy as jnp
from jax import lax
from jax.experimental import pallas as pl
from jax.experimental.pallas import tpu as pltpu
from jax.experimental.pallas import tpu_sc as plsc

B = 2
N = 20000
NP = 20480
NT = 16
CH = NP // NT
NV = CH // 16
TAIL = N - (NT - 1) * CH
TVPAD = (CH - TAIL) // 16
K_PRE = 1000
K_POST = 100
CAND = 1024
IMG = 512.0
NMS_T = 0.5
SCORE_T = 0.05
MIN_SIZE = 0.001
CLIP = math.log(1000.0 / 16.0)
NEG = -1e30


def _scalar(x):
    return jnp.max(x) if getattr(x, "ndim", 0) else x


def _sc_body(logits_hbm, anchors_hbm, deltas_hbm,
             outb_hbm, outs_hbm,
             logit_v, key_v, hist2_v, allhist_v, rhist_v, cnt16_v, cntbuf_v,
             lgt_v, leq_v, anch_v, delt_v, stage_v, myc_v, soa_v,
             pub_v, nmsbuf_v, outb_v, outs_v,
             sh_hist, sh_cnt, sh_cand, sh_nms):
    s = lax.axis_index("s")
    b = lax.axis_index("c")
    iota = lax.iota(jnp.int32, 16)

    @pl.when(s < NT - 1)
    def _():
        pltpu.sync_copy(logits_hbm.at[pl.ds(b * N + s * CH, CH)], logit_v)

    @pl.when(s == NT - 1)
    def _():
        pltpu.sync_copy(logits_hbm.at[pl.ds(b * N + s * CH, TAIL)],
                        logit_v.at[pl.ds(0, TAIL)])
        negv = jnp.full((16,), NEG, jnp.float32)

        def fill_body(i, _):
            logit_v[pl.ds(TAIL + i * 16, 16)] = negv
            return 0

        lax.fori_loop(0, TVPAD, fill_body, 0)

    def key_body(i, _):
        x = logit_v[pl.ds(i * 16, 16)]
        ub = plsc.bitcast(x, jnp.uint32)
        neg = (ub >> jnp.uint32(31)) != jnp.uint32(0)
        k = jnp.where(neg, ~ub, ub | jnp.uint32(0x80000000))
        key_v[pl.ds(i * 16, 16)] = k
        return 0

    lax.fori_loop(0, NV, key_body, 0)

    prefix = jnp.uint32(0)
    rank = jnp.int32(K_PRE)
    ones_i = jnp.full((16,), 1, jnp.int32)
    zeros_i = jnp.zeros((16,), jnp.int32)

    for rnd in range(4):
        shift = 24 - 8 * rnd
        pmask_c = jnp.uint32((0xFFFFFFFF << (shift + 8)) & 0xFFFFFFFF) if rnd else None

        def zero_body(j, _):
            for t in range(16):
                hist2_v[t, pl.ds(j * 16, 16)] = zeros_i
            return 0

        lax.fori_loop(0, 16, zero_body, 0)

        def hist_body(i, _):
            k = key_v[pl.ds(i * 16, 16)]
            bucket = lax.convert_element_type(
                (k >> jnp.uint32(shift)) & jnp.uint32(0xFF), jnp.int32)
            if rnd == 0:
                plsc.addupdate_scatter(hist2_v, [iota, bucket], ones_i)
            else:
                m = (k & pmask_c) == (prefix & pmask_c)
                plsc.addupdate_scatter(hist2_v, [iota, bucket], ones_i, mask=m)
            return 0

        lax.fori_loop(0, NV, hist_body, 0)

        def red_body(j, _):
            acc = hist2_v[0, pl.ds(j * 16, 16)]
            for t in range(1, 16):
                acc = acc + hist2_v[t, pl.ds(j * 16, 16)]
            rhist_v[pl.ds(j * 16, 16)] = acc
            return 0

        lax.fori_loop(0, 16, red_body, 0)

        pltpu.sync_copy(rhist_v, sh_hist.at[s])
        plsc.subcore_barrier()
        pltpu.sync_copy(sh_hist, allhist_v)
        plsc.subcore_barrier()

        def red2_body(j, _):
            acc = allhist_v[0, pl.ds(j * 16, 16)]
            for t in range(1, 16):
                acc = acc + allhist_v[t, pl.ds(j * 16, 16)]
            rhist_v[pl.ds(j * 16, 16)] = acc
            return 0

        lax.fori_loop(0, 16, red2_body, 0)

        gsum = []
        for v in range(16):
            gsum.append(jnp.sum(rhist_v[pl.ds(v * 16, 16)]))
        suf = jnp.int32(0)
        g_acc = jnp.int32(0)
        above_acc = jnp.int32(0)
        for v in range(15, -1, -1):
            hit = (suf < rank) & (suf + gsum[v] >= rank)
            g_acc = jnp.where(hit, jnp.int32(v), g_acc)
            above_acc = jnp.where(hit, suf, above_acc)
            suf = suf + gsum[v]
        rank = rank - above_acc
        gvec = rhist_v[pl.ds(g_acc * 16, 16)]
        lval = []
        for l in range(16):
            lval.append(jnp.sum(jnp.where(iota == l, gvec, 0)))
        suf = jnp.int32(0)
        l_acc = jnp.int32(0)
        above_acc = jnp.int32(0)
        for l in range(15, -1, -1):
            hit = (suf < rank) & (suf + lval[l] >= rank)
            l_acc = jnp.where(hit, jnp.int32(l), l_acc)
            above_acc = jnp.where(hit, suf, above_acc)
            suf = suf + lval[l]
        rank = rank - above_acc
        beta = g_acc * 16 + l_acc
        prefix = prefix | (lax.convert_element_type(beta, jnp.uint32)
                           << jnp.uint32(shift))

    thresh = prefix
    need_eq = rank

    def comp_body(i, carry):
        gt_pos, eq_pos = carry
        k = key_v[pl.ds(i * 16, 16)]
        gidx = s * CH + i * 16 + iota
        mgt = k > thresh
        meq = k == thresh
        plsc.store_compressed(lgt_v.at[pl.ds(gt_pos, 16)], gidx, mask=mgt)
        plsc.store_compressed(leq_v.at[pl.ds(eq_pos, 16)], gidx, mask=meq)
        gt_pos = gt_pos + _scalar(plsc.all_reduce_population_count(mgt))
        eq_pos = eq_pos + _scalar(plsc.all_reduce_population_count(meq))
        return (gt_pos, eq_pos)

    gt_cnt, eq_cnt = lax.fori_loop(0, NV, comp_body, (jnp.int32(0), jnp.int32(0)))

    cnt16_v[...] = jnp.where(iota == 0, gt_cnt, jnp.where(iota == 1, eq_cnt, 0))
    pltpu.sync_copy(cnt16_v, sh_cnt.at[s])

    for c in range(8):
        plsc.store_scatter(stage_v, [iota, jnp.full((16,), c, jnp.int32)],
                           jnp.full((16,), NEG if c == 5 else 0.0, jnp.float32))
    pad_pos = K_PRE + lax.rem(s + iota, jnp.int32(CAND - K_PRE))
    pltpu.sync_copy(stage_v, sh_cand.at[pad_pos])

    plsc.subcore_barrier()
    pltpu.sync_copy(sh_cnt, cntbuf_v)

    gt_counts = plsc.load_gather(cntbuf_v, [iota, zeros_i])
    eq_counts = plsc.load_gather(cntbuf_v, [iota, ones_i])
    total_gt = jnp.sum(gt_counts)
    gt_before = jnp.sum(jnp.where(iota < s, gt_counts, 0))
    eq_before = jnp.sum(jnp.where(iota < s, eq_counts, 0))

    @pl.when(s < NT - 1)
    def _():
        pltpu.sync_copy(anchors_hbm.at[pl.ds(s * CH, CH), :], anch_v)
        pltpu.sync_copy(deltas_hbm.at[b, pl.ds(s * CH, CH), :], delt_v)

    @pl.when(s == NT - 1)
    def _():
        pltpu.sync_copy(anchors_hbm.at[pl.ds(s * CH, TAIL), :],
                        anch_v.at[pl.ds(0, TAIL), :])
        pltpu.sync_copy(deltas_hbm.at[b, pl.ds(s * CH, TAIL), :],
                        delt_v.at[pl.ds(0, TAIL), :])

    zeros_f = jnp.zeros((16,), jnp.float32)

    def process_chunk(idxv, posv):
        lidx = idxv - s * CH

        def col(ref, c):
            return plsc.load_gather(ref, [lidx, jnp.full((16,), c, jnp.int32)])

        ax1, ay1, ax2, ay2 = col(anch_v, 0), col(anch_v, 1), col(anch_v, 2), col(anch_v, 3)
        dx, dy, dw, dh = col(delt_v, 0), col(delt_v, 1), col(delt_v, 2), col(delt_v, 3)
        aw = ax2 - ax1
        ah = ay2 - ay1
        cx = ax1 + 0.5 * aw
        cy = ay1 + 0.5 * ah
        dw = jnp.minimum(dw, CLIP)
        dh = jnp.minimum(dh, CLIP)
        pcx = dx * aw + cx
        pcy = dy * ah + cy
        pw = jnp.exp(dw) * aw
        ph = jnp.exp(dh) * ah
        x1 = jnp.clip(pcx - 0.5 * pw, 0.0, IMG)
        y1 = jnp.clip(pcy - 0.5 * ph, 0.0, IMG)
        x2 = jnp.clip(pcx + 0.5 * pw, 0.0, IMG)
        y2 = jnp.clip(pcy + 0.5 * ph, 0.0, IMG)
        okbox = ((x2 - x1) >= MIN_SIZE) & ((y2 - y1) >= MIN_SIZE)
        lg = plsc.load_gather(logit_v, [idxv - s * CH])
        sig = 1.0 / (1.0 + jnp.exp(-lg))
        sc = jnp.where(okbox & (sig > SCORE_T), sig, 0.0)
        dumped = posv >= K_PRE
        sc = jnp.where(dumped, 0.0, sc)
        lg = jnp.where(dumped, NEG, lg)
        for c, val in enumerate((x1, y1, x2, y2, sc, lg, zeros_f, zeros_f)):
            plsc.store_scatter(stage_v, [iota, jnp.full((16,), c, jnp.int32)], val)
        pltpu.sync_copy(stage_v, sh_cand.at[posv])

    dump_row = K_PRE + s

    def gt_body(k, _):
        j = k * 16 + iota
        valid = j < gt_cnt
        idxv = jnp.where(valid, lgt_v[pl.ds(k * 16, 16)], s * CH)
        posv = jnp.where(valid, gt_before + j, dump_row)
        process_chunk(idxv, posv)
        return 0

    lax.fori_loop(0, (gt_cnt + 15) // 16, gt_body, 0)

    def eq_body(k, _):
        j = k * 16 + iota
        valid = j < eq_cnt
        idxv = jnp.where(valid, leq_v[pl.ds(k * 16, 16)], s * CH)
        r = eq_before + j
        posv = jnp.where(valid & (r < need_eq), total_gt + r, dump_row)
        process_chunk(idxv, posv)
        return 0

    lax.fori_loop(0, (eq_cnt + 15) // 16, eq_body, 0)

    plsc.subcore_barrier()

    pltpu.sync_copy(sh_cand.at[pl.ds(s * 64, 64)], myc_v)

    score_vecs = []
    logit_vecs = []
    for r in range(4):
        rows = r * 16 + iota
        for c in range(4):
            v = plsc.load_gather(myc_v, [rows, jnp.full((16,), c, jnp.int32)])
            soa_v[c, pl.ds(r * 16, 16)] = v
        x1 = soa_v[0, pl.ds(r * 16, 16)]
        y1 = soa_v[1, pl.ds(r * 16, 16)]
        x2 = soa_v[2, pl.ds(r * 16, 16)]
        y2 = soa_v[3, pl.ds(r * 16, 16)]
        soa_v[4, pl.ds(r * 16, 16)] = (x2 - x1) * (y2 - y1)
        score_vecs.append(plsc.load_gather(myc_v, [rows, jnp.full((16,), 4, jnp.int32)]))
        logit_vecs.append(plsc.load_gather(myc_v, [rows, jnp.full((16,), 5, jnp.int32)]))

    def local_pick(vecs):
        m01 = jnp.maximum(vecs[0], vecs[1])
        m23 = jnp.maximum(vecs[2], vecs[3])
        m = jnp.max(jnp.maximum(m01, m23))
        big = jnp.int32(1 << 30)
        argp = jnp.int32(1 << 30)
        for r in range(4):
            pr = jnp.min(jnp.where(vecs[r] == m, s * 64 + r * 16 + iota, big))
            argp = jnp.minimum(argp, pr)
        return m, argp

    def fetch_box(argp):
        lp = jnp.full((16,), argp - s * 64, jnp.int32)
        return [plsc.load_gather(soa_v, [jnp.full((16,), c, jnp.int32), lp])
                for c in range(4)]

    def publish_and_reduce(m, box, argp):
        row = jnp.full((16,), m, jnp.float32)
        for c in range(4):
            row = jnp.where(iota == c + 1, box[c], row)
        row = jnp.where(iota == 5,
                        lax.convert_element_type(argp, jnp.float32), row)
        pub_v[...] = row
        pltpu.sync_copy(pub_v, sh_nms.at[s])
        plsc.subcore_barrier()
        pltpu.sync_copy(sh_nms, nmsbuf_v)
        plsc.subcore_barrier()
        mall = plsc.load_gather(nmsbuf_v, [iota, zeros_i])
        gmax = jnp.max(mall)
        wt = jnp.min(jnp.where(mall == gmax, iota, 99))
        wtv = jnp.full((16,), wt, jnp.int32)

        def get(c):
            return plsc.load_gather(nmsbuf_v, [wtv, jnp.full((16,), c, jnp.int32)])

        wbox = [get(1), get(2), get(3), get(4)]
        wargp = lax.convert_element_type(jnp.max(get(5)), jnp.int32)
        return gmax, wbox, wargp

    pm, pargp = local_pick(logit_vecs)
    pbox = fetch_box(pargp)
    _, pad_box, _ = publish_and_reduce(pm, pbox, pargp)

    outs_v[pl.ds(96, 16)] = jnp.where(iota < 4, outs_v[pl.ds(96, 16)], 0.0)

    def nms_body(i, carry):
        sv = list(carry[0:4])
        pb = list(carry[4:8])
        m, argp = local_pick(sv)
        box = fetch_box(argp)
        gmax, wbox, wargp = publish_and_reduce(m, box, argp)
        empty = gmax <= 0.0
        ob = [jnp.where(empty, pb[c], wbox[c]) for c in range(4)]

        @pl.when(s == 0)
        def _():
            val = ob[0]
            for c in range(1, 4):
                val = jnp.where(iota == c, ob[c], val)
            plsc.store_scatter(outb_v, [i * 4 + iota],
                               val, mask=iota < 4)
            plsc.store_scatter(outs_v, [jnp.full((16,), i, jnp.int32)],
                               jnp.full((16,), gmax, jnp.float32),
                               mask=iota < 1)

        wx1, wy1, wx2, wy2 = wbox
        a1 = (wx2 - wx1) * (wy2 - wy1)
        for r in range(4):
            bx1 = soa_v[0, pl.ds(r * 16, 16)]
            by1 = soa_v[1, pl.ds(r * 16, 16)]
            bx2 = soa_v[2, pl.ds(r * 16, 16)]
            by2 = soa_v[3, pl.ds(r * 16, 16)]
            a2 = soa_v[4, pl.ds(r * 16, 16)]
            ix1 = jnp.maximum(wx1, bx1)
            iy1 = jnp.maximum(wy1, by1)
            ix2 = jnp.minimum(wx2, bx2)
            iy2 = jnp.minimum(wy2, by2)
            inter = jnp.maximum(ix2 - ix1, 0.0) * jnp.maximum(iy2 - iy1, 0.0)
            iou = inter / (a1 + a2 - inter + 1e-8)
            gpos = s * 64 + r * 16 + iota
            sv[r] = jnp.where((iou > NMS_T) | (gpos == wargp), 0.0, sv[r])
        return tuple(sv) + tuple(pb)

    lax.fori_loop(0, K_POST, nms_body, tuple(score_vecs) + tuple(pad_box))

    @pl.when(s == 0)
    def _():
        pltpu.sync_copy(outb_v, outb_hbm.at[pl.ds(b * K_POST * 4, K_POST * 4)])
        pltpu.sync_copy(outs_v, outs_hbm.at[pl.ds(b * 112, 112)])


@jax.jit
def _run(logits, anchors_flat, deltas_flat):
    mesh = plsc.VectorSubcoreMesh(core_axis_name="c", subcore_axis_name="s")
    f = pl.kernel(
        _sc_body,
        out_type=(
            jax.ShapeDtypeStruct((B * K_POST * 4,), jnp.float32),
            jax.ShapeDtypeStruct((B * 112,), jnp.float32),
        ),
        mesh=mesh,
        compiler_params=pltpu.CompilerParams(needs_layout_passes=False, use_tc_tiling_on_sc=False),
        scratch_types=[
            pltpu.VMEM((CH,), jnp.float32),
            pltpu.VMEM((CH,), jnp.uint32),
            pltpu.VMEM((16, 256), jnp.int32),
            pltpu.VMEM((16, 256), jnp.int32),
            pltpu.VMEM((256,), jnp.int32),
            pltpu.VMEM((16,), jnp.int32),
            pltpu.VMEM((16, 16), jnp.int32),
            pltpu.VMEM((CH + 16,), jnp.int32),
            pltpu.VMEM((CH + 16,), jnp.int32),
            pltpu.VMEM((CH, 4), jnp.float32),
            pltpu.VMEM((CH, 4), jnp.float32),
            pltpu.VMEM((16, 8), jnp.float32),
            pltpu.VMEM((64, 8), jnp.float32),
            pltpu.VMEM((5, 64), jnp.float32),
            pltpu.VMEM((16,), jnp.float32),
            pltpu.VMEM((16, 16), jnp.float32),
            pltpu.VMEM((K_POST * 4,), jnp.float32),
            pltpu.VMEM((112,), jnp.float32),
            pltpu.VMEM_SHARED((16, 256), jnp.int32),
            pltpu.VMEM_SHARED((16, 16), jnp.int32),
            pltpu.VMEM_SHARED((CAND, 8), jnp.float32),
            pltpu.VMEM_SHARED((16, 16), jnp.float32),
        ],
    )
    return f(logits, anchors_flat, deltas_flat)


def kernel(pred_class, pred_bbox_deltas, anchors):
    logits = pred_class.reshape(B * N)
    boxes_flat, scores_flat = _run(logits, anchors, pred_bbox_deltas)
    sel_boxes = boxes_flat.reshape(B, K_POST, 4)
    sel_scores = scores_flat.reshape(B, 112)[:, :K_POST]
    return sel_boxes, sel_scores


# speedup vs baseline: 1.1955x; 1.1955x over previous
import functools
import math

import jax
import jax.numpy as jnp
from jax import lax
from jax.experimental import pallas as pl
from jax.experimental.pallas import tpu as pltpu
from jax.experimental.pallas import tpu_sc as plsc

B = 2
N = 20000
NP = 20480
NT = 16
CH = NP // NT
NV = CH // 16
TAIL = N - (NT - 1) * CH
TVPAD = (CH - TAIL) // 16
K_PRE = 1000
K_POST = 100
CAND = 1024
IMG = 512.0
NMS_T = 0.5
SCORE_T = 0.05
MIN_SIZE = 0.001
CLIP = math.log(1000.0 / 16.0)
NEG = -1e30


def _scalar(x):
    return jnp.max(x) if getattr(x, "ndim", 0) else x


def _sc_body(logits_hbm, anchors_hbm, deltas_hbm,
             outb_hbm, outs_hbm,
             logit_v, key_v, hist2_v, allhist_v, rhist_v, cnt16_v, cntbuf_v,
             lgt_v, leq_v, anch_v, delt_v, stage_v, myc_v, soa_v,
             pub_v, nmsbuf_v, outb_v, outs_v,
             sh_hist, sh_cnt, sh_cand, sh_nms):
    s = lax.axis_index("s")
    b = lax.axis_index("c")
    iota = lax.iota(jnp.int32, 16)

    @pl.when(s < NT - 1)
    def _():
        pltpu.sync_copy(logits_hbm.at[pl.ds(b * N + s * CH, CH)], logit_v)

    @pl.when(s == NT - 1)
    def _():
        pltpu.sync_copy(logits_hbm.at[pl.ds(b * N + s * CH, TAIL)],
                        logit_v.at[pl.ds(0, TAIL)])
        negv = jnp.full((16,), NEG, jnp.float32)

        def fill_body(i, _):
            logit_v[pl.ds(TAIL + i * 16, 16)] = negv
            return 0

        lax.fori_loop(0, TVPAD, fill_body, 0)

    def key_body(i, _):
        x = logit_v[pl.ds(i * 16, 16)]
        ub = plsc.bitcast(x, jnp.uint32)
        neg = (ub >> jnp.uint32(31)) != jnp.uint32(0)
        k = jnp.where(neg, ~ub, ub | jnp.uint32(0x80000000))
        key_v[pl.ds(i * 16, 16)] = k
        return 0

    lax.fori_loop(0, NV, key_body, 0)

    prefix = jnp.uint32(0)
    rank = jnp.int32(K_PRE)
    ones_i = jnp.full((16,), 1, jnp.int32)
    zeros_i = jnp.zeros((16,), jnp.int32)

    for rnd in range(4):
        shift = 24 - 8 * rnd
        pmask_c = jnp.uint32((0xFFFFFFFF << (shift + 8)) & 0xFFFFFFFF) if rnd else None

        def zero_body(j, _):
            for t in range(16):
                hist2_v[t, pl.ds(j * 16, 16)] = zeros_i
            return 0

        lax.fori_loop(0, 16, zero_body, 0)

        def hist_body(i, _):
            k = key_v[pl.ds(i * 16, 16)]
            bucket = lax.convert_element_type(
                (k >> jnp.uint32(shift)) & jnp.uint32(0xFF), jnp.int32)
            if rnd == 0:
                plsc.addupdate_scatter(hist2_v, [iota, bucket], ones_i)
            else:
                m = (k & pmask_c) == (prefix & pmask_c)
                plsc.addupdate_scatter(hist2_v, [iota, bucket], ones_i, mask=m)
            return 0

        lax.fori_loop(0, NV, hist_body, 0)

        def red_body(j, _):
            acc = hist2_v[0, pl.ds(j * 16, 16)]
            for t in range(1, 16):
                acc = acc + hist2_v[t, pl.ds(j * 16, 16)]
            rhist_v[pl.ds(j * 16, 16)] = acc
            return 0

        lax.fori_loop(0, 16, red_body, 0)

        pltpu.sync_copy(rhist_v, sh_hist.at[s])
        plsc.subcore_barrier()
        pltpu.sync_copy(sh_hist, allhist_v)
        plsc.subcore_barrier()

        def red2_body(j, _):
            acc = allhist_v[0, pl.ds(j * 16, 16)]
            for t in range(1, 16):
                acc = acc + allhist_v[t, pl.ds(j * 16, 16)]
            rhist_v[pl.ds(j * 16, 16)] = acc
            return 0

        lax.fori_loop(0, 16, red2_body, 0)

        gsum = []
        for v in range(16):
            gsum.append(jnp.sum(rhist_v[pl.ds(v * 16, 16)]))
        suf = jnp.int32(0)
        g_acc = jnp.int32(0)
        above_acc = jnp.int32(0)
        for v in range(15, -1, -1):
            hit = (suf < rank) & (suf + gsum[v] >= rank)
            g_acc = jnp.where(hit, jnp.int32(v), g_acc)
            above_acc = jnp.where(hit, suf, above_acc)
            suf = suf + gsum[v]
        rank = rank - above_acc
        gvec = rhist_v[pl.ds(g_acc * 16, 16)]
        lval = []
        for l in range(16):
            lval.append(jnp.sum(jnp.where(iota == l, gvec, 0)))
        suf = jnp.int32(0)
        l_acc = jnp.int32(0)
        above_acc = jnp.int32(0)
        for l in range(15, -1, -1):
            hit = (suf < rank) & (suf + lval[l] >= rank)
            l_acc = jnp.where(hit, jnp.int32(l), l_acc)
            above_acc = jnp.where(hit, suf, above_acc)
            suf = suf + lval[l]
        rank = rank - above_acc
        beta = g_acc * 16 + l_acc
        prefix = prefix | (lax.convert_element_type(beta, jnp.uint32)
                           << jnp.uint32(shift))

    thresh = prefix
    need_eq = rank

    def comp_body(i, carry):
        gt_pos, eq_pos = carry
        k = key_v[pl.ds(i * 16, 16)]
        gidx = s * CH + i * 16 + iota
        mgt = k > thresh
        meq = k == thresh
        plsc.store_compressed(lgt_v.at[pl.ds(gt_pos, 16)], gidx, mask=mgt)
        plsc.store_compressed(leq_v.at[pl.ds(eq_pos, 16)], gidx, mask=meq)
        gt_pos = gt_pos + _scalar(plsc.all_reduce_population_count(mgt))
        eq_pos = eq_pos + _scalar(plsc.all_reduce_population_count(meq))
        return (gt_pos, eq_pos)

    gt_cnt, eq_cnt = lax.fori_loop(0, NV, comp_body, (jnp.int32(0), jnp.int32(0)))

    cnt16_v[...] = jnp.where(iota == 0, gt_cnt, jnp.where(iota == 1, eq_cnt, 0))
    pltpu.sync_copy(cnt16_v, sh_cnt.at[s])

    for c in range(8):
        plsc.store_scatter(stage_v, [iota, jnp.full((16,), c, jnp.int32)],
                           jnp.full((16,), NEG if c == 5 else 0.0, jnp.float32))
    pad_pos = K_PRE + lax.rem(s + iota, jnp.int32(CAND - K_PRE))
    pltpu.sync_copy(stage_v, sh_cand.at[pad_pos])

    plsc.subcore_barrier()
    pltpu.sync_copy(sh_cnt, cntbuf_v)

    gt_counts = plsc.load_gather(cntbuf_v, [iota, zeros_i])
    eq_counts = plsc.load_gather(cntbuf_v, [iota, ones_i])
    total_gt = jnp.sum(gt_counts)
    gt_before = jnp.sum(jnp.where(iota < s, gt_counts, 0))
    eq_before = jnp.sum(jnp.where(iota < s, eq_counts, 0))

    @pl.when(s < NT - 1)
    def _():
        pltpu.sync_copy(anchors_hbm.at[pl.ds(s * CH * 4, CH * 4)], anch_v)
        pltpu.sync_copy(deltas_hbm.at[pl.ds((b * N + s * CH) * 4, CH * 4)],
                        delt_v)

    @pl.when(s == NT - 1)
    def _():
        pltpu.sync_copy(anchors_hbm.at[pl.ds(s * CH * 4, TAIL * 4)],
                        anch_v.at[pl.ds(0, TAIL * 4)])
        pltpu.sync_copy(deltas_hbm.at[pl.ds((b * N + s * CH) * 4, TAIL * 4)],
                        delt_v.at[pl.ds(0, TAIL * 4)])

    zeros_f = jnp.zeros((16,), jnp.float32)

    def process_chunk(idxv, posv):
        base = (idxv - s * CH) * 4

        def col(ref, c):
            return plsc.load_gather(ref, [base + c])

        ax1, ay1, ax2, ay2 = col(anch_v, 0), col(anch_v, 1), col(anch_v, 2), col(anch_v, 3)
        dx, dy, dw, dh = col(delt_v, 0), col(delt_v, 1), col(delt_v, 2), col(delt_v, 3)
        aw = ax2 - ax1
        ah = ay2 - ay1
        cx = ax1 + 0.5 * aw
        cy = ay1 + 0.5 * ah
        dw = jnp.minimum(dw, CLIP)
        dh = jnp.minimum(dh, CLIP)
        pcx = dx * aw + cx
        pcy = dy * ah + cy
        pw = jnp.exp(dw) * aw
        ph = jnp.exp(dh) * ah
        x1 = jnp.clip(pcx - 0.5 * pw, 0.0, IMG)
        y1 = jnp.clip(pcy - 0.5 * ph, 0.0, IMG)
        x2 = jnp.clip(pcx + 0.5 * pw, 0.0, IMG)
        y2 = jnp.clip(pcy + 0.5 * ph, 0.0, IMG)
        okbox = ((x2 - x1) >= MIN_SIZE) & ((y2 - y1) >= MIN_SIZE)
        lg = plsc.load_gather(logit_v, [idxv - s * CH])
        sig = 1.0 / (1.0 + jnp.exp(-lg))
        sc = jnp.where(okbox & (sig > SCORE_T), sig, 0.0)
        dumped = posv >= K_PRE
        sc = jnp.where(dumped, 0.0, sc)
        lg = jnp.where(dumped, NEG, lg)
        for c, val in enumerate((x1, y1, x2, y2, sc, lg, zeros_f, zeros_f)):
            plsc.store_scatter(stage_v, [iota, jnp.full((16,), c, jnp.int32)], val)
        pltpu.sync_copy(stage_v, sh_cand.at[posv])

    dump_row = K_PRE + s

    def gt_body(k, _):
        j = k * 16 + iota
        valid = j < gt_cnt
        idxv = jnp.where(valid, lgt_v[pl.ds(k * 16, 16)], s * CH)
        posv = jnp.where(valid, gt_before + j, dump_row)
        process_chunk(idxv, posv)
        return 0

    lax.fori_loop(0, (gt_cnt + 15) // 16, gt_body, 0)

    def eq_body(k, _):
        j = k * 16 + iota
        valid = j < eq_cnt
        idxv = jnp.where(valid, leq_v[pl.ds(k * 16, 16)], s * CH)
        r = eq_before + j
        posv = jnp.where(valid & (r < need_eq), total_gt + r, dump_row)
        process_chunk(idxv, posv)
        return 0

    lax.fori_loop(0, (eq_cnt + 15) // 16, eq_body, 0)

    plsc.subcore_barrier()

    pltpu.sync_copy(sh_cand.at[pl.ds(s * 64, 64)], myc_v)

    score_vecs = []
    logit_vecs = []
    for r in range(4):
        rows = r * 16 + iota
        for c in range(4):
            v = plsc.load_gather(myc_v, [rows, jnp.full((16,), c, jnp.int32)])
            soa_v[c, pl.ds(r * 16, 16)] = v
        x1 = soa_v[0, pl.ds(r * 16, 16)]
        y1 = soa_v[1, pl.ds(r * 16, 16)]
        x2 = soa_v[2, pl.ds(r * 16, 16)]
        y2 = soa_v[3, pl.ds(r * 16, 16)]
        soa_v[4, pl.ds(r * 16, 16)] = (x2 - x1) * (y2 - y1)
        score_vecs.append(plsc.load_gather(myc_v, [rows, jnp.full((16,), 4, jnp.int32)]))
        logit_vecs.append(plsc.load_gather(myc_v, [rows, jnp.full((16,), 5, jnp.int32)]))

    def local_pick(vecs):
        m01 = jnp.maximum(vecs[0], vecs[1])
        m23 = jnp.maximum(vecs[2], vecs[3])
        m = jnp.max(jnp.maximum(m01, m23))
        big = jnp.int32(1 << 30)
        argp = jnp.int32(1 << 30)
        for r in range(4):
            pr = jnp.min(jnp.where(vecs[r] == m, s * 64 + r * 16 + iota, big))
            argp = jnp.minimum(argp, pr)
        return m, argp

    def fetch_box(argp):
        lp = jnp.full((16,), argp - s * 64, jnp.int32)
        return [plsc.load_gather(soa_v, [jnp.full((16,), c, jnp.int32), lp])
                for c in range(4)]

    def publish_and_reduce(m, box, argp, p):
        row = jnp.full((16,), m, jnp.float32)
        for c in range(4):
            row = jnp.where(iota == c + 1, box[c], row)
        row = jnp.where(iota == 5,
                        lax.convert_element_type(argp, jnp.float32), row)
        pub_v[...] = row
        pltpu.sync_copy(pub_v, sh_nms.at[p, s])
        plsc.subcore_barrier()
        pltpu.sync_copy(sh_nms.at[p], nmsbuf_v)
        mall = plsc.load_gather(nmsbuf_v, [iota, zeros_i])
        gmax = jnp.max(mall)
        wt = jnp.min(jnp.where(mall == gmax, iota, 99))
        wtv = jnp.full((16,), wt, jnp.int32)

        def get(c):
            return plsc.load_gather(nmsbuf_v, [wtv, jnp.full((16,), c, jnp.int32)])

        wbox = [get(1), get(2), get(3), get(4)]
        wargp = lax.convert_element_type(jnp.max(get(5)), jnp.int32)
        return gmax, wbox, wargp

    pm, pargp = local_pick(logit_vecs)
    pbox = fetch_box(pargp)
    _, pad_box, _ = publish_and_reduce(pm, pbox, pargp, jnp.int32(0))

    outs_v[pl.ds(96, 16)] = jnp.where(iota < 4, outs_v[pl.ds(96, 16)], 0.0)

    def nms_body(i, carry):
        sv = list(carry[0:4])
        pb = list(carry[4:8])
        m, argp = local_pick(sv)
        box = fetch_box(argp)
        gmax, wbox, wargp = publish_and_reduce(m, box, argp,
                                               lax.rem(i + 1, jnp.int32(2)))
        empty = gmax <= 0.0
        ob = [jnp.where(empty, pb[c], wbox[c]) for c in range(4)]

        @pl.when(s == 0)
        def _():
            val = ob[0]
            for c in range(1, 4):
                val = jnp.where(iota == c, ob[c], val)
            plsc.store_scatter(outb_v, [i * 4 + iota],
                               val, mask=iota < 4)
            plsc.store_scatter(outs_v, [jnp.full((16,), i, jnp.int32)],
                               jnp.full((16,), gmax, jnp.float32),
                               mask=iota < 1)

        wx1, wy1, wx2, wy2 = wbox
        a1 = (wx2 - wx1) * (wy2 - wy1)
        for r in range(4):
            bx1 = soa_v[0, pl.ds(r * 16, 16)]
            by1 = soa_v[1, pl.ds(r * 16, 16)]
            bx2 = soa_v[2, pl.ds(r * 16, 16)]
            by2 = soa_v[3, pl.ds(r * 16, 16)]
            a2 = soa_v[4, pl.ds(r * 16, 16)]
            ix1 = jnp.maximum(wx1, bx1)
            iy1 = jnp.maximum(wy1, by1)
            ix2 = jnp.minimum(wx2, bx2)
            iy2 = jnp.minimum(wy2, by2)
            inter = jnp.maximum(ix2 - ix1, 0.0) * jnp.maximum(iy2 - iy1, 0.0)
            iou = inter / (a1 + a2 - inter + 1e-8)
            gpos = s * 64 + r * 16 + iota
            sv[r] = jnp.where((iou > NMS_T) | (gpos == wargp), 0.0, sv[r])
        return tuple(sv) + tuple(pb)

    lax.fori_loop(0, K_POST, nms_body, tuple(score_vecs) + tuple(pad_box))

    @pl.when(s == 0)
    def _():
        pltpu.sync_copy(outb_v, outb_hbm.at[pl.ds(b * K_POST * 4, K_POST * 4)])
        pltpu.sync_copy(outs_v, outs_hbm.at[pl.ds(b * 112, 112)])


@jax.jit
def _run(logits, anchors_flat, deltas_flat):
    mesh = plsc.VectorSubcoreMesh(core_axis_name="c", subcore_axis_name="s")
    f = pl.kernel(
        _sc_body,
        out_type=(
            jax.ShapeDtypeStruct((B * K_POST * 4,), jnp.float32),
            jax.ShapeDtypeStruct((B * 112,), jnp.float32),
        ),
        mesh=mesh,
        compiler_params=pltpu.CompilerParams(needs_layout_passes=False, use_tc_tiling_on_sc=False),
        scratch_types=[
            pltpu.VMEM((CH,), jnp.float32),
            pltpu.VMEM((CH,), jnp.uint32),
            pltpu.VMEM((16, 256), jnp.int32),
            pltpu.VMEM((16, 256), jnp.int32),
            pltpu.VMEM((256,), jnp.int32),
            pltpu.VMEM((16,), jnp.int32),
            pltpu.VMEM((16, 16), jnp.int32),
            pltpu.VMEM((CH + 16,), jnp.int32),
            pltpu.VMEM((CH + 16,), jnp.int32),
            pltpu.VMEM((CH * 4,), jnp.float32),
            pltpu.VMEM((CH * 4,), jnp.float32),
            pltpu.VMEM((16, 8), jnp.float32),
            pltpu.VMEM((64, 8), jnp.float32),
            pltpu.VMEM((5, 64), jnp.float32),
            pltpu.VMEM((16,), jnp.float32),
            pltpu.VMEM((16, 16), jnp.float32),
            pltpu.VMEM((K_POST * 4,), jnp.float32),
            pltpu.VMEM((112,), jnp.float32),
            pltpu.VMEM_SHARED((16, 256), jnp.int32),
            pltpu.VMEM_SHARED((16, 16), jnp.int32),
            pltpu.VMEM_SHARED((CAND, 8), jnp.float32),
            pltpu.VMEM_SHARED((2, 16, 16), jnp.float32),
        ],
    )
    return f(logits, anchors_flat, deltas_flat)


def kernel(pred_class, pred_bbox_deltas, anchors):
    logits = pred_class.reshape(B * N)
    anchors_flat = anchors.reshape(N * 4)
    deltas_flat = pred_bbox_deltas.reshape(B * N * 4)
    boxes_flat, scores_flat = _run(logits, anchors_flat, deltas_flat)
    sel_boxes = boxes_flat.reshape(B, K_POST, 4)
    sel_scores = scores_flat.reshape(B, 112)[:, :K_POST]
    return sel_boxes, sel_scores
